# Initial kernel scaffold; baseline (speedup 1.0000x reference)
import jax
import jax.numpy as jnp
from jax.experimental import pallas as pl


def kernel(x, edge_index, W1, b1, W2, b2):
    raise NotImplementedError("write your pallas kernel here")


# speedup vs baseline: 6.1735x; 6.1735x over previous
import functools

import jax
import jax.numpy as jnp
from jax import lax
from jax.experimental import pallas as pl
from jax.experimental.pallas import tpu as pltpu
from jax.experimental.pallas import tpu_sc as plsc

N = 10000
E = 160000
D = 256
DH = 128
DA = 136
NC = 2
NS = 16
K = 80
EPW = E // NS
NCHUNK = EPW // K
NP = 10240
RPS = NP // NS

_sc_mesh = plsc.VectorSubcoreMesh(
    core_axis_name="c", subcore_axis_name="s", num_cores=NC, num_subcores=NS
)


@functools.partial(
    pl.kernel,
    out_type=(
        jax.ShapeDtypeStruct((NP, DA), jnp.float32),
        jax.ShapeDtypeStruct((NP, DA), jnp.float32),
    ),
    mesh=_sc_mesh,
    scratch_types=[
        pltpu.VMEM((NCHUNK, K), jnp.int32),
        pltpu.VMEM((NCHUNK, K), jnp.int32),
        pltpu.VMEM((K, DA), jnp.float32),
        pltpu.VMEM((K, DA), jnp.float32),
        pltpu.VMEM_SHARED((NP, DA), jnp.float32),
        pltpu.SemaphoreType.DMA,
        pltpu.SemaphoreType.DMA,
    ],
    compiler_params=pltpu.CompilerParams(use_tc_tiling_on_sc=False),
)
def _sc_aggregate(xaug, colsrc, rows2d, zeros, out0, out1,
                  colv, rowv, g0, g1, acc, s0, s1):
    c = lax.axis_index("c")
    s = lax.axis_index("s")
    sl = pl.ds(s * RPS, RPS)

    pltpu.sync_copy(zeros, acc.at[sl])
    pltpu.sync_copy(colsrc.at[s], colv)
    pltpu.sync_copy(rows2d.at[s], rowv)
    plsc.subcore_barrier()

    xtab = xaug.at[c]

    def fire(i, buf, sem):
        pltpu.async_copy(xtab.at[colv.at[i]], buf, sem)

    def drain_scatter(i, buf, sem):
        pltpu.make_async_copy(xtab.at[colv.at[i]], buf, sem).wait()
        pltpu.sync_copy(buf, acc.at[rowv.at[i]], add=True)

    fire(0, g0, s0)

    def step(i, carry):
        @pl.when(i % 2 == 0)
        def _():
            @pl.when(i + 1 < NCHUNK)
            def _():
                fire(i + 1, g1, s1)
            drain_scatter(i, g0, s0)

        @pl.when(i % 2 == 1)
        def _():
            @pl.when(i + 1 < NCHUNK)
            def _():
                fire(i + 1, g0, s0)
            drain_scatter(i, g1, s1)

        return carry

    lax.fori_loop(0, NCHUNK, step, 0)
    plsc.subcore_barrier()

    @pl.when(c == 0)
    def _():
        pltpu.sync_copy(acc.at[sl], out0.at[sl])

    @pl.when(c == 1)
    def _():
        pltpu.sync_copy(acc.at[sl], out1.at[sl])


R = 1000


def _tc_mlp(o0, o1, w1t, b1, w2t, b2, out):
    inv = 1.0 / jnp.maximum(o0[:, DH:DH + 1], 1.0)
    a0 = o0[:, :DH] * inv
    a1 = o1[:, :DH] * inv
    h = jnp.dot(a0, w1t[:DH, :], preferred_element_type=jnp.float32)
    h = h + jnp.dot(a1, w1t[DH:, :], preferred_element_type=jnp.float32)
    h = jnp.maximum(h + b1[...], 0.0)
    out[...] = jnp.dot(h, w2t[...], preferred_element_type=jnp.float32) + b2[...]


_tc_call = pl.pallas_call(
    _tc_mlp,
    grid=(N // R,),
    in_specs=[
        pl.BlockSpec((R, DA), lambda i: (i, 0)),
        pl.BlockSpec((R, DA), lambda i: (i, 0)),
        pl.BlockSpec((D, D), lambda i: (0, 0)),
        pl.BlockSpec((1, D), lambda i: (0, 0)),
        pl.BlockSpec((D, D), lambda i: (0, 0)),
        pl.BlockSpec((1, D), lambda i: (0, 0)),
    ],
    out_specs=pl.BlockSpec((R, D), lambda i: (i, 0)),
    out_shape=jax.ShapeDtypeStruct((N, D), jnp.float32),
)


def kernel(x, edge_index, W1, b1, W2, b2):
    row = edge_index[0].astype(jnp.int32)
    col = edge_index[1].astype(jnp.int32)

    xh = jnp.moveaxis(x.reshape(N, NC, DH), 1, 0)
    pad = jnp.zeros((NC, N, DA - DH), x.dtype).at[:, :, 0].set(1.0)
    xaug = jnp.concatenate([xh, pad], axis=2)

    colsrc = col.reshape(NS, NCHUNK, K)
    rows2d = row.reshape(NS, NCHUNK, K)
    zeros = jnp.zeros((RPS, DA), x.dtype)

    out0, out1 = _sc_aggregate(xaug, colsrc, rows2d, zeros)
    return _tc_call(out0, out1, W1.T, b1.reshape(1, D), W2.T, b2.reshape(1, D))


# speedup vs baseline: 8.3563x; 1.3536x over previous
import functools

import jax
import jax.numpy as jnp
from jax import lax
from jax.experimental import pallas as pl
from jax.experimental.pallas import tpu as pltpu
from jax.experimental.pallas import tpu_sc as plsc

N = 10000
E = 160000
D = 256
DH = 128
CW = 8
NC = 2
NS = 16
K = 80
EPW = E // NS
NCHUNK = EPW // K
NP = 10240
RPS = NP // NS

_sc_mesh = plsc.VectorSubcoreMesh(
    core_axis_name="c", subcore_axis_name="s", num_cores=NC, num_subcores=NS
)


@functools.partial(
    pl.kernel,
    out_type=(
        jax.ShapeDtypeStruct((NP, DH), jnp.float32),
        jax.ShapeDtypeStruct((NP, DH), jnp.float32),
        jax.ShapeDtypeStruct((NP, CW), jnp.float32),
        jax.ShapeDtypeStruct((NP, CW), jnp.float32),
    ),
    mesh=_sc_mesh,
    scratch_types=[
        pltpu.VMEM((NCHUNK, K), jnp.int32),
        pltpu.VMEM((NCHUNK, K), jnp.int32),
        pltpu.VMEM((K, DH), jnp.float32),
        pltpu.VMEM((K, DH), jnp.float32),
        pltpu.VMEM((K, CW), jnp.float32),
        pltpu.VMEM_SHARED((NP, DH), jnp.float32),
        pltpu.VMEM_SHARED((NP, CW), jnp.float32),
        pltpu.SemaphoreType.DMA,
        pltpu.SemaphoreType.DMA,
    ],
    compiler_params=pltpu.CompilerParams(use_tc_tiling_on_sc=False),
)
def _sc_aggregate(x0, x1, colsrc, rows2d, zeros, zeros8, ones8,
                  outa, outb, cnta, cntb,
                  colv, rowv, g0, g1, ones_v, acc, cnt, s0, s1):
    c = lax.axis_index("c")
    s = lax.axis_index("s")
    sl = pl.ds(s * RPS, RPS)
    do_cnt = (c == 0) == (s < NS // 2)

    pltpu.sync_copy(zeros, acc.at[sl])
    pltpu.sync_copy(zeros8, cnt.at[sl])
    pltpu.sync_copy(colsrc.at[s], colv)
    pltpu.sync_copy(rows2d.at[s], rowv)
    pltpu.sync_copy(ones8, ones_v)
    plsc.subcore_barrier()

    def fire(i, buf, sem, xtab):
        pltpu.async_copy(xtab.at[colv.at[i]], buf, sem)

    def fire2(i, buf, sem):
        @pl.when(c == 0)
        def _():
            fire(i, buf, sem, x0)

        @pl.when(c == 1)
        def _():
            fire(i, buf, sem, x1)

    def drain_scatter(i, buf, sem):
        pltpu.make_async_copy(x0.at[colv.at[i]], buf, sem).wait()
        pltpu.sync_copy(buf, acc.at[rowv.at[i]], add=True)

        @pl.when(do_cnt)
        def _():
            pltpu.sync_copy(ones_v, cnt.at[rowv.at[i]], add=True)

    fire2(0, g0, s0)

    def step(i, carry):
        @pl.when(i % 2 == 0)
        def _():
            @pl.when(i + 1 < NCHUNK)
            def _():
                fire2(i + 1, g1, s1)
            drain_scatter(i, g0, s0)

        @pl.when(i % 2 == 1)
        def _():
            @pl.when(i + 1 < NCHUNK)
            def _():
                fire2(i + 1, g0, s0)
            drain_scatter(i, g1, s1)

        return carry

    lax.fori_loop(0, NCHUNK, step, 0)
    plsc.subcore_barrier()

    @pl.when(c == 0)
    def _():
        pltpu.sync_copy(acc.at[sl], outa.at[sl])
        pltpu.sync_copy(cnt.at[sl], cnta.at[sl])

    @pl.when(c == 1)
    def _():
        pltpu.sync_copy(acc.at[sl], outb.at[sl])
        pltpu.sync_copy(cnt.at[sl], cntb.at[sl])


R = 1000


def _tc_mlp(o0, o1, ca, cb, w1t, b1, w2t, b2, out):
    inv = 1.0 / jnp.maximum(ca[:, :1] + cb[:, :1], 1.0)
    a0 = o0[...] * inv
    a1 = o1[...] * inv
    h = jnp.dot(a0, w1t[:DH, :], preferred_element_type=jnp.float32)
    h = h + jnp.dot(a1, w1t[DH:, :], preferred_element_type=jnp.float32)
    h = jnp.maximum(h + b1[...], 0.0)
    out[...] = jnp.dot(h, w2t[...], preferred_element_type=jnp.float32) + b2[...]


_tc_call = pl.pallas_call(
    _tc_mlp,
    grid=(N // R,),
    in_specs=[
        pl.BlockSpec((R, DH), lambda i: (i, 0)),
        pl.BlockSpec((R, DH), lambda i: (i, 0)),
        pl.BlockSpec((R, CW), lambda i: (i, 0)),
        pl.BlockSpec((R, CW), lambda i: (i, 0)),
        pl.BlockSpec((D, D), lambda i: (0, 0)),
        pl.BlockSpec((1, D), lambda i: (0, 0)),
        pl.BlockSpec((D, D), lambda i: (0, 0)),
        pl.BlockSpec((1, D), lambda i: (0, 0)),
    ],
    out_specs=pl.BlockSpec((R, D), lambda i: (i, 0)),
    out_shape=jax.ShapeDtypeStruct((N, D), jnp.float32),
)


def kernel(x, edge_index, W1, b1, W2, b2):
    row = edge_index[0].astype(jnp.int32)
    col = edge_index[1].astype(jnp.int32)

    x0 = x[:, :DH]
    x1 = x[:, DH:]
    colsrc = col.reshape(NS, NCHUNK, K)
    rows2d = row.reshape(NS, NCHUNK, K)
    zeros = jnp.zeros((RPS, DH), x.dtype)
    zeros8 = jnp.zeros((RPS, CW), x.dtype)
    ones8 = jnp.ones((K, CW), x.dtype)

    outa, outb, cnta, cntb = _sc_aggregate(
        x0, x1, colsrc, rows2d, zeros, zeros8, ones8)
    return _tc_call(outa, outb, cnta, cntb,
                    W1.T, b1.reshape(1, D), W2.T, b2.reshape(1, D))
